# baseline (speedup 1.0000x reference)
import functools

import jax
import jax.numpy as jnp
from jax import lax
from jax.experimental import pallas as pl
from jax.experimental.pallas import tpu as pltpu
from jax.experimental.pallas import tpu_sc as plsc

N = 10000
E = 320000
D = 128
B = 16

NC = 2
NS = 16
K = 125
EPW = E // (NC * NS)
NCHUNK = EPW // K
NSTAGE = 5
SLAB = NCHUNK // NSTAGE
CH = 400
NCH = N // CH


def _sc_spmm(x, ei3):
    mesh = plsc.VectorSubcoreMesh(core_axis_name="c", subcore_axis_name="s")

    @functools.partial(
        pl.kernel,
        out_type=(
            jax.ShapeDtypeStruct((N, D), jnp.float32),
            jax.ShapeDtypeStruct((N, D), jnp.float32),
        ),
        mesh=mesh,
        scratch_types=[
            pltpu.VMEM((SLAB, K), jnp.int32),
            pltpu.VMEM((SLAB, K), jnp.int32),
            pltpu.VMEM((SLAB, K), jnp.int32),
            pltpu.VMEM((SLAB, K), jnp.int32),
            pltpu.VMEM((K, D), jnp.float32),
            pltpu.VMEM((K, D), jnp.float32),
            pltpu.VMEM_SHARED((N, D), jnp.float32),
            pltpu.SemaphoreType.DMA,
            pltpu.SemaphoreType.DMA,
            pltpu.SemaphoreType.DMA,
            pltpu.SemaphoreType.DMA,
        ],
    )
    def spmm(x_hbm, ei_hbm, out0, out1, srcb0, dstb0, srcb1, dstb1,
             rows0, rows1, acc, gsem0, gsem1, isem, psem):
        c = lax.axis_index("c")
        s = lax.axis_index("s")
        src_hbm = ei_hbm.at[0]
        dst_hbm = ei_hbm.at[1]

        for t in range(2):
            j = s + NS * t
            @pl.when(j < NCH)
            def _():
                r0 = pl.multiple_of(j * CH, CH)
                pltpu.async_copy(x_hbm.at[pl.ds(r0, CH)], acc.at[pl.ds(r0, CH)],
                                 isem)

        rb = pl.multiple_of((c * NS + s) * NCHUNK, NCHUNK)

        slabs = [(srcb0, dstb0), (srcb1, dstb1)]
        sb0 = pl.multiple_of(rb, SLAB)
        pltpu.sync_copy(src_hbm.at[pl.ds(sb0, SLAB)], srcb0)
        pltpu.sync_copy(dst_hbm.at[pl.ds(sb0, SLAB)], dstb0)

        for stage in range(NSTAGE):
            srcb, dstb = slabs[stage % 2]
            if stage + 1 < NSTAGE:
                nsrcb, ndstb = slabs[(stage + 1) % 2]
                sbn = pl.multiple_of(rb + (stage + 1) * SLAB, SLAB)
                pltpu.async_copy(src_hbm.at[pl.ds(sbn, SLAB)], nsrcb, psem)
                pltpu.async_copy(dst_hbm.at[pl.ds(sbn, SLAB)], ndstb, psem)

            pltpu.async_copy(x_hbm.at[srcb.at[0]], rows0, gsem0)
            pltpu.async_copy(x_hbm.at[srcb.at[1]], rows1, gsem1)

            if stage == 0:
                for t in range(2):
                    j = s + NS * t
                    @pl.when(j < NCH)
                    def _():
                        r0 = pl.multiple_of(j * CH, CH)
                        pltpu.make_async_copy(x_hbm.at[pl.ds(r0, CH)],
                                              acc.at[pl.ds(r0, CH)], isem).wait()
                plsc.subcore_barrier()

            def step(i, carry):
                j0 = 2 * i
                j1 = 2 * i + 1
                pltpu.make_async_copy(x_hbm.at[srcb.at[j0]], rows0, gsem0).wait()
                pltpu.sync_copy(rows0, acc.at[dstb.at[j0]], add=True)

                @pl.when(i < SLAB // 2 - 1)
                def _():
                    pltpu.async_copy(x_hbm.at[srcb.at[j0 + 2]], rows0, gsem0)

                pltpu.make_async_copy(x_hbm.at[srcb.at[j1]], rows1, gsem1).wait()
                pltpu.sync_copy(rows1, acc.at[dstb.at[j1]], add=True)

                @pl.when(i < SLAB // 2 - 1)
                def _():
                    pltpu.async_copy(x_hbm.at[srcb.at[j1 + 2]], rows1, gsem1)

                return carry

            lax.fori_loop(0, SLAB // 2, step, 0)

            if stage + 1 < NSTAGE:
                pltpu.make_async_copy(src_hbm.at[pl.ds(sbn, SLAB)], nsrcb,
                                      psem).wait()
                pltpu.make_async_copy(dst_hbm.at[pl.ds(sbn, SLAB)], ndstb,
                                      psem).wait()

        plsc.subcore_barrier()

        for t in range(2):
            j = s + NS * t
            @pl.when((j < NCH) & (c == 0))
            def _():
                r0 = pl.multiple_of(j * CH, CH)
                pltpu.sync_copy(acc.at[pl.ds(r0, CH)], out0.at[pl.ds(r0, CH)])

            @pl.when((j < NCH) & (c == 1))
            def _():
                r0 = pl.multiple_of(j * CH, CH)
                pltpu.sync_copy(acc.at[pl.ds(r0, CH)], out1.at[pl.ds(r0, CH)])

    return spmm(x, ei3)


BLK = 2000
NBLK = N // BLK


def _tc_body(x_ref, a0_ref, a1_ref, bat_ref, eps_ref, w_ref, bias_ref,
             out_ref, pooled_ref, seg_acc, segb_acc):
    i = pl.program_id(0)
    eps = eps_ref[0, 0]
    out = (eps - 1.0) * x_ref[...] + a0_ref[...] + a1_ref[...]
    out = jnp.maximum(out, 0.0)
    out_ref[...] = out

    onehot = (bat_ref[...] ==
              lax.broadcasted_iota(jnp.int32, (BLK, B), 1)).astype(jnp.float32)
    seg_part = lax.dot_general(onehot, out, (((0,), (0,)), ((), ())),
                               preferred_element_type=jnp.float32)
    bias_b = jnp.broadcast_to(bias_ref[...], (BLK, D))
    segb_part = lax.dot_general(onehot, bias_b, (((0,), (0,)), ((), ())),
                                preferred_element_type=jnp.float32)

    @pl.when(i == 0)
    def _():
        seg_acc[...] = jnp.zeros_like(seg_acc)
        segb_acc[...] = jnp.zeros_like(segb_acc)

    seg_acc[...] += seg_part
    segb_acc[...] += segb_part

    @pl.when(i == NBLK - 1)
    def _():
        pooled_ref[...] = lax.dot_general(
            seg_acc[...], w_ref[...], (((1,), (1,)), ((), ())),
            preferred_element_type=jnp.float32) + segb_acc[...]


def _tc_fuse(x, agg0, agg1, batch2d, eps2d, W, bias2d):
    return pl.pallas_call(
        _tc_body,
        grid=(NBLK,),
        in_specs=[
            pl.BlockSpec((BLK, D), lambda i: (i, 0)),
            pl.BlockSpec((BLK, D), lambda i: (i, 0)),
            pl.BlockSpec((BLK, D), lambda i: (i, 0)),
            pl.BlockSpec((BLK, 1), lambda i: (i, 0)),
            pl.BlockSpec(memory_space=pltpu.SMEM),
            pl.BlockSpec((D, D), lambda i: (0, 0)),
            pl.BlockSpec((1, D), lambda i: (0, 0)),
        ],
        out_specs=[
            pl.BlockSpec((BLK, D), lambda i: (i, 0)),
            pl.BlockSpec((B, D), lambda i: (0, 0)),
        ],
        out_shape=[
            jax.ShapeDtypeStruct((N, D), jnp.float32),
            jax.ShapeDtypeStruct((B, D), jnp.float32),
        ],
        scratch_shapes=[
            pltpu.VMEM((B, D), jnp.float32),
            pltpu.VMEM((B, D), jnp.float32),
        ],
    )(x, agg0, agg1, batch2d, eps2d, W, bias2d)


def kernel(x, edge_index, batch, eps, W, b):
    agg0, agg1 = _sc_spmm(x, edge_index.reshape(2, E // K, K))
    out, pooled2 = _tc_fuse(x, agg0, agg1,
                            batch.reshape(N, 1), eps.reshape(1, 1),
                            W, b.reshape(1, D))
    return (out, pooled2)


# speedup vs baseline: 1.0346x; 1.0346x over previous
import functools

import jax
import jax.numpy as jnp
from jax import lax
from jax.experimental import pallas as pl
from jax.experimental.pallas import tpu as pltpu
from jax.experimental.pallas import tpu_sc as plsc

N = 10000
E = 320000
D = 128
B = 16

NC = 2
NS = 16
K = 125
EPW = E // (NC * NS)
NCHUNK = EPW // K
NSTAGE = 2
SLAB = NCHUNK // NSTAGE
CH = 400
NCH = N // CH


def _sc_spmm(x, ei3):
    mesh = plsc.VectorSubcoreMesh(core_axis_name="c", subcore_axis_name="s")

    @functools.partial(
        pl.kernel,
        out_type=(
            jax.ShapeDtypeStruct((N, D), jnp.float32),
            jax.ShapeDtypeStruct((N, D), jnp.float32),
        ),
        mesh=mesh,
        scratch_types=[
            pltpu.VMEM((SLAB, K), jnp.int32),
            pltpu.VMEM((SLAB, K), jnp.int32),
            pltpu.VMEM((K, D), jnp.float32),
            pltpu.VMEM((K, D), jnp.float32),
            pltpu.VMEM_SHARED((N, D), jnp.float32),
            pltpu.SemaphoreType.DMA,
            pltpu.SemaphoreType.DMA,
            pltpu.SemaphoreType.DMA,
        ],
    )
    def spmm(x_hbm, ei_hbm, out0, out1, srcb, dstb,
             rows0, rows1, acc, gsem0, gsem1, isem):
        c = lax.axis_index("c")
        s = lax.axis_index("s")
        src_hbm = ei_hbm.at[0]
        dst_hbm = ei_hbm.at[1]

        for t in range(2):
            j = s + NS * t
            @pl.when(j < NCH)
            def _():
                r0 = pl.multiple_of(j * CH, CH)
                pltpu.async_copy(x_hbm.at[pl.ds(r0, CH)], acc.at[pl.ds(r0, CH)],
                                 isem)

        rb = pl.multiple_of((c * NS + s) * NCHUNK, NCHUNK)

        for stage in range(NSTAGE):
            sb = pl.multiple_of(rb + stage * SLAB, SLAB)
            pltpu.sync_copy(src_hbm.at[pl.ds(sb, SLAB)], srcb)
            pltpu.sync_copy(dst_hbm.at[pl.ds(sb, SLAB)], dstb)

            pltpu.async_copy(x_hbm.at[srcb.at[0]], rows0, gsem0)
            pltpu.async_copy(x_hbm.at[srcb.at[1]], rows1, gsem1)

            if stage == 0:
                for t in range(2):
                    j = s + NS * t
                    @pl.when(j < NCH)
                    def _():
                        r0 = pl.multiple_of(j * CH, CH)
                        pltpu.make_async_copy(x_hbm.at[pl.ds(r0, CH)],
                                              acc.at[pl.ds(r0, CH)], isem).wait()
                plsc.subcore_barrier()

            def step(i, carry):
                j0 = 2 * i
                j1 = 2 * i + 1
                pltpu.make_async_copy(x_hbm.at[srcb.at[j0]], rows0, gsem0).wait()
                pltpu.sync_copy(rows0, acc.at[dstb.at[j0]], add=True)

                @pl.when(i < SLAB // 2 - 1)
                def _():
                    pltpu.async_copy(x_hbm.at[srcb.at[j0 + 2]], rows0, gsem0)

                pltpu.make_async_copy(x_hbm.at[srcb.at[j1]], rows1, gsem1).wait()
                pltpu.sync_copy(rows1, acc.at[dstb.at[j1]], add=True)

                @pl.when(i < SLAB // 2 - 1)
                def _():
                    pltpu.async_copy(x_hbm.at[srcb.at[j1 + 2]], rows1, gsem1)

                return carry

            lax.fori_loop(0, SLAB // 2, step, 0)

        plsc.subcore_barrier()

        for t in range(2):
            j = s + NS * t
            @pl.when((j < NCH) & (c == 0))
            def _():
                r0 = pl.multiple_of(j * CH, CH)
                pltpu.sync_copy(acc.at[pl.ds(r0, CH)], out0.at[pl.ds(r0, CH)])

            @pl.when((j < NCH) & (c == 1))
            def _():
                r0 = pl.multiple_of(j * CH, CH)
                pltpu.sync_copy(acc.at[pl.ds(r0, CH)], out1.at[pl.ds(r0, CH)])

    return spmm(x, ei3)


BLK = 2000
NBLK = N // BLK


def _tc_body(x_ref, a0_ref, a1_ref, bat_ref, eps_ref, w_ref, bias_ref,
             out_ref, pooled_ref, seg_acc, segb_acc):
    i = pl.program_id(0)
    eps = eps_ref[0, 0]
    out = (eps - 1.0) * x_ref[...] + a0_ref[...] + a1_ref[...]
    out = jnp.maximum(out, 0.0)
    out_ref[...] = out

    onehot = (bat_ref[...] ==
              lax.broadcasted_iota(jnp.int32, (BLK, B), 1)).astype(jnp.float32)
    seg_part = lax.dot_general(onehot, out, (((0,), (0,)), ((), ())),
                               preferred_element_type=jnp.float32)
    bias_b = jnp.broadcast_to(bias_ref[...], (BLK, D))
    segb_part = lax.dot_general(onehot, bias_b, (((0,), (0,)), ((), ())),
                                preferred_element_type=jnp.float32)

    @pl.when(i == 0)
    def _():
        seg_acc[...] = jnp.zeros_like(seg_acc)
        segb_acc[...] = jnp.zeros_like(segb_acc)

    seg_acc[...] += seg_part
    segb_acc[...] += segb_part

    @pl.when(i == NBLK - 1)
    def _():
        pooled_ref[...] = lax.dot_general(
            seg_acc[...], w_ref[...], (((1,), (1,)), ((), ())),
            preferred_element_type=jnp.float32) + segb_acc[...]


def _tc_fuse(x, agg0, agg1, batch2d, eps2d, W, bias2d):
    return pl.pallas_call(
        _tc_body,
        grid=(NBLK,),
        in_specs=[
            pl.BlockSpec((BLK, D), lambda i: (i, 0)),
            pl.BlockSpec((BLK, D), lambda i: (i, 0)),
            pl.BlockSpec((BLK, D), lambda i: (i, 0)),
            pl.BlockSpec((BLK, 1), lambda i: (i, 0)),
            pl.BlockSpec(memory_space=pltpu.SMEM),
            pl.BlockSpec((D, D), lambda i: (0, 0)),
            pl.BlockSpec((1, D), lambda i: (0, 0)),
        ],
        out_specs=[
            pl.BlockSpec((BLK, D), lambda i: (i, 0)),
            pl.BlockSpec((B, D), lambda i: (0, 0)),
        ],
        out_shape=[
            jax.ShapeDtypeStruct((N, D), jnp.float32),
            jax.ShapeDtypeStruct((B, D), jnp.float32),
        ],
        scratch_shapes=[
            pltpu.VMEM((B, D), jnp.float32),
            pltpu.VMEM((B, D), jnp.float32),
        ],
    )(x, agg0, agg1, batch2d, eps2d, W, bias2d)


def kernel(x, edge_index, batch, eps, W, b):
    agg0, agg1 = _sc_spmm(x, edge_index.reshape(2, E // K, K))
    out, pooled2 = _tc_fuse(x, agg0, agg1,
                            batch.reshape(N, 1), eps.reshape(1, 1),
                            W, b.reshape(1, D))
    return (out, pooled2)
